# baseline (speedup 1.0000x reference)
import functools

import jax
import jax.numpy as jnp
from jax import lax
from jax.experimental import pallas as pl
from jax.experimental.pallas import tpu as pltpu
from jax.experimental.pallas import tpu_sc as plsc

_N = 10000
_E = 320000
_HID = 128
_NC, _NS, _L = 2, 16, 16
_NW = _NC * _NS
_CH1 = 125
_CPW1 = _E // _CH1 // _NW
_CH2 = 80
_CPW2 = _E // _CH2 // _NW
_RPS = _N // _NS

_MESH = plsc.VectorSubcoreMesh(
    core_axis_name="c", subcore_axis_name="s", num_cores=_NC, num_subcores=_NS)
_SC_PARAMS = pltpu.CompilerParams(use_tc_tiling_on_sc=False)

_f32 = jnp.float32


def _zero_accum(zbuf, nrows, accum, s):
    zeros = jnp.zeros((_L,), _f32)

    def zb(r, _):
        for q in range(_HID // _L):
            zbuf[r, pl.ds(q * _L, _L)] = zeros
        return 0

    lax.fori_loop(0, nrows, zb, 0, unroll=2)
    nfull, rem = _RPS // nrows, _RPS % nrows
    for t in range(nfull):
        pltpu.sync_copy(zbuf, accum.at[pl.ds(s * _RPS + t * nrows, nrows)])
    if rem:
        pltpu.sync_copy(zbuf.at[pl.ds(0, rem)],
                        accum.at[pl.ds(s * _RPS + nfull * nrows, rem)])


@functools.partial(
    pl.kernel,
    out_type=jax.ShapeDtypeStruct((_NC, _N, _HID), _f32),
    mesh=_MESH,
    compiler_params=_SC_PARAMS,
    scratch_types=[
        pltpu.VMEM((_CH1, _HID), _f32),
        pltpu.VMEM((_CH1, _HID), _f32),
        pltpu.VMEM((_CH1,), jnp.int32),
        pltpu.VMEM((_CH1,), jnp.int32),
        pltpu.VMEM_SHARED((_N, _HID), _f32),
        pltpu.SemaphoreType.DMA,
        pltpu.SemaphoreType.DMA,
    ],
)
def _sc_segsum(e_hbm, idx_hbm, out_hbm, rows0, rows1, idx0, idx1,
               accum, insem0, insem1):
    c = lax.axis_index("c")
    s = lax.axis_index("s")
    _zero_accum(rows0, _CH1, accum, s)
    plsc.subcore_barrier()

    rows = (rows0, rows1)
    idxs = (idx0, idx1)
    insems = (insem0, insem1)
    base = (c * _NS + s) * _CPW1

    def in_copies(k, b):
        return (
            pltpu.make_async_copy(
                e_hbm.at[1, pl.ds(k * _CH1, _CH1), :], rows[b], insems[b]),
            pltpu.make_async_copy(idx_hbm.at[k], idxs[b], insems[b]),
        )

    for cp in in_copies(base, 0):
        cp.start()

    def step(q, b):
        k = base + q
        for cp in in_copies(k, b):
            cp.wait()

        @pl.when(q + 1 < _CPW1)
        def _():
            for cp in in_copies(k + 1, 1 - b):
                cp.start()

        pltpu.sync_copy(rows[b], accum.at[idxs[b]], add=True)

    def outer(t, _):
        step(t * 2, 0)
        step(t * 2 + 1, 1)
        return 0

    lax.fori_loop(0, _CPW1 // 2, outer, 0)

    plsc.subcore_barrier()
    sl = pl.ds(s * _RPS, _RPS)
    pltpu.sync_copy(accum.at[sl], out_hbm.at[c, sl, :])


@functools.partial(
    pl.kernel,
    out_type=jax.ShapeDtypeStruct((_NC, _N, _HID), _f32),
    mesh=_MESH,
    compiler_params=_SC_PARAMS,
    scratch_types=[
        pltpu.VMEM((_CH2, _HID), _f32),
        pltpu.VMEM((_CH2, _HID), _f32),
        pltpu.VMEM((_CH2, _HID), _f32),
        pltpu.VMEM((_CH2, _HID), _f32),
        pltpu.VMEM((_CH2,), jnp.int32),
        pltpu.VMEM((_CH2,), jnp.int32),
        pltpu.VMEM((_CH2,), jnp.int32),
        pltpu.VMEM((_CH2,), jnp.int32),
        pltpu.VMEM_SHARED((_N, _HID), _f32),
        pltpu.SemaphoreType.DMA,
        pltpu.SemaphoreType.DMA,
        pltpu.SemaphoreType.DMA,
        pltpu.SemaphoreType.DMA,
    ],
)
def _sc_gather_mul_segsum(wh_hbm, jdx_hbm, idst_hbm, tab_hbm, out_hbm,
                          wrows0, wrows1, grows0, grows1, jdx0, jdx1,
                          ddx0, ddx1, accum, insem0, insem1, gsem0, gsem1):
    c = lax.axis_index("c")
    s = lax.axis_index("s")
    _zero_accum(wrows0, _CH2, accum, s)
    plsc.subcore_barrier()

    wrows = (wrows0, wrows1)
    grows = (grows0, grows1)
    jdxs = (jdx0, jdx1)
    ddxs = (ddx0, ddx1)
    insems = (insem0, insem1)
    gsems = (gsem0, gsem1)
    base = (c * _NS + s) * _CPW2

    def in_copies(k, b):
        return (
            pltpu.make_async_copy(
                wh_hbm.at[pl.ds(k * _CH2, _CH2), :], wrows[b], insems[b]),
            pltpu.make_async_copy(jdx_hbm.at[k], jdxs[b], insems[b]),
            pltpu.make_async_copy(idst_hbm.at[k], ddxs[b], insems[b]),
        )

    def gath(b):
        return pltpu.make_async_copy(tab_hbm.at[jdxs[b]], grows[b], gsems[b])

    def mul(b):
        def body(r, _):
            for qq in range(_HID // _L):
                sl2 = pl.ds(qq * _L, _L)
                wrows[b][r, sl2] = wrows[b][r, sl2] * grows[b][r, sl2]
            return 0

        lax.fori_loop(0, _CH2, body, 0, unroll=4)

    def step(q, b):
        k = base + q

        @pl.when(q + 1 < _CPW2)
        def _():
            for cp in in_copies(k + 1, 1 - b):
                cp.start()

        gath(b).wait()

        @pl.when(q + 1 < _CPW2)
        def _():
            for cp in in_copies(k + 1, 1 - b):
                cp.wait()
            pltpu.async_copy(tab_hbm.at[jdxs[1 - b]], grows[1 - b],
                             gsems[1 - b])

        mul(b)
        pltpu.sync_copy(wrows[b], accum.at[ddxs[b]], add=True)

    for cp in in_copies(base, 0):
        cp.start()
    for cp in in_copies(base, 0):
        cp.wait()
    pltpu.async_copy(tab_hbm.at[jdxs[0]], grows[0], gsems[0])
    step(0, 0)

    def outer(t, _):
        step(t * 2 + 1, 1)
        step(t * 2 + 2, 0)
        return 0

    lax.fori_loop(0, (_CPW2 - 1) // 2, outer, 0)

    plsc.subcore_barrier()
    sl = pl.ds(s * _RPS, _RPS)
    pltpu.sync_copy(accum.at[sl], out_hbm.at[c, sl, :])


def _silu(x):
    return x * (1.0 / (1.0 + jnp.exp(-x)))


def _edge_mlp_body(x_ref, w0_ref, b0_ref, w1_ref, b1_ref, o_ref):
    h = jnp.dot(x_ref[...], w0_ref[...], preferred_element_type=_f32) + b0_ref[...]
    h = _silu(h)
    o_ref[...] = jnp.dot(h, w1_ref[...], preferred_element_type=_f32) + b1_ref[...]


def _edge_mlp(fea, w0, b0, w1, b1):
    blk = 2000
    grid = (_E // blk,)
    return pl.pallas_call(
        _edge_mlp_body,
        grid=grid,
        in_specs=[
            pl.BlockSpec((blk, 16), lambda g: (g, 0)),
            pl.BlockSpec((16, 128), lambda g: (0, 0)),
            pl.BlockSpec((128,), lambda g: (0,)),
            pl.BlockSpec((128, 128), lambda g: (0, 0)),
            pl.BlockSpec((128,), lambda g: (0,)),
        ],
        out_specs=pl.BlockSpec((blk, 128), lambda g: (g, 0)),
        out_shape=jax.ShapeDtypeStruct((_E, 128), _f32),
    )(fea, w0, b0, w1, b1)


def _vh_body(p_ref, w_ref, v_ref, vh_ref):
    v = p_ref[0] + p_ref[1]
    v_ref[...] = v
    vh_ref[...] = jnp.dot(v, w_ref[...], preferred_element_type=_f32)


def _v_vhull(parts, w):
    blk = 2000
    grid = (_N // blk,)
    return pl.pallas_call(
        _vh_body,
        grid=grid,
        in_specs=[
            pl.BlockSpec((_NC, blk, 128), lambda g: (0, g, 0)),
            pl.BlockSpec((128, 128), lambda g: (0, 0)),
        ],
        out_specs=[
            pl.BlockSpec((blk, 128), lambda g: (g, 0)),
            pl.BlockSpec((blk, 128), lambda g: (g, 0)),
        ],
        out_shape=[
            jax.ShapeDtypeStruct((_N, 128), _f32),
            jax.ShapeDtypeStruct((_N, 128), _f32),
        ],
    )(parts, w)


def _tail_body(v_ref, p_ref, w1h, b1h, w2h, b2h, wcv, wco, bc, wu, bu,
               wl0, bl0, wl1, bl1, wo, o_ref):
    oh = p_ref[0] + p_ref[1]
    oh = _silu(jnp.dot(oh, w1h[...], preferred_element_type=_f32) + b1h[...])
    oh = jnp.dot(oh, w2h[...], preferred_element_type=_f32) + b2h[...]
    v = v_ref[...]
    t = _silu(jnp.dot(v, wcv[...], preferred_element_type=_f32)
              + jnp.dot(oh, wco[...], preferred_element_type=_f32) + bc[...])
    u = jnp.dot(t, wu[...], preferred_element_type=_f32) + bu[...]
    u = _silu(jnp.dot(u, wl0[...], preferred_element_type=_f32) + bl0[...])
    u = _silu(jnp.dot(u, wl1[...], preferred_element_type=_f32) + bl1[...])
    o_ref[...] = jnp.dot(u, wo[...], preferred_element_type=_f32)


def _tail(v, parts, W_lin1h, b_lin1h, W_lin2h, b_lin2h, W_cat, b_cat,
          W_up, b_up, W_l0, b_l0, W_l1, b_l1, W_out):
    blk = 2000
    grid = (_N // blk,)
    wcv = W_cat[:128]
    wco = W_cat[128:]
    full = lambda *shape: pl.BlockSpec(shape, lambda g: (0,) * len(shape))
    return pl.pallas_call(
        _tail_body,
        grid=grid,
        in_specs=[
            pl.BlockSpec((blk, 128), lambda g: (g, 0)),
            pl.BlockSpec((_NC, blk, 128), lambda g: (0, g, 0)),
            full(128, 128), full(128,), full(128, 256), full(256,),
            full(128, 128), full(256, 128), full(128,),
            full(128, 256), full(256,),
            full(256, 256), full(256,), full(256, 256), full(256,),
            full(256, 128),
        ],
        out_specs=pl.BlockSpec((blk, 128), lambda g: (g, 0)),
        out_shape=jax.ShapeDtypeStruct((_N, 128), _f32),
    )(v, parts, W_lin1h, b_lin1h, W_lin2h, b_lin2h, wcv, wco, b_cat,
      W_up, b_up, W_l0, b_l0, W_l1, b_l1, W_out)


def kernel(e, i, fea_hull, edge_index_hull, W_lin_hull, W_mlp0, b_mlp0,
           W_mlp1, b_mlp1, W_lin1h, b_lin1h, W_lin2h, b_lin2h, W_cat, b_cat,
           W_up, b_up, W_l0, b_l0, W_l1, b_l1, W_out):
    i_r = i.astype(jnp.int32).reshape(_E // _CH1, _CH1)
    eih = edge_index_hull.astype(jnp.int32).reshape(2, _E // _CH2, _CH2)

    v_parts = _sc_segsum(e, i_r)
    W_h = _edge_mlp(fea_hull, W_mlp0, b_mlp0, W_mlp1, b_mlp1)
    v, v_hull = _v_vhull(v_parts, W_lin_hull)
    oh_parts = _sc_gather_mul_segsum(W_h, eih[0], eih[1], v_hull)
    return _tail(v, oh_parts, W_lin1h, b_lin1h, W_lin2h, b_lin2h,
                 W_cat, b_cat, W_up, b_up, W_l0, b_l0, W_l1, b_l1, W_out)


# speedup vs baseline: 1.3596x; 1.3596x over previous
import functools

import jax
import jax.numpy as jnp
from jax import lax
from jax.experimental import pallas as pl
from jax.experimental.pallas import tpu as pltpu
from jax.experimental.pallas import tpu_sc as plsc

_N = 10000
_E = 320000
_HID = 128
_NC, _NS, _L = 2, 16, 16
_NW = _NC * _NS
_CH1 = 100
_CPW1 = _E // _CH1 // _NW
_CH2 = 100
_CPW2 = _E // _CH2 // _NW
_RPS = _N // _NS

_MESH = plsc.VectorSubcoreMesh(
    core_axis_name="c", subcore_axis_name="s", num_cores=_NC, num_subcores=_NS)
_SC_PARAMS = pltpu.CompilerParams(use_tc_tiling_on_sc=False)

_f32 = jnp.float32


def _zero_accum(zbuf, nrows, accum, s):
    zeros = jnp.zeros((_L,), _f32)

    def zb(r, _):
        for q in range(_HID // _L):
            zbuf[r, pl.ds(q * _L, _L)] = zeros
        return 0

    lax.fori_loop(0, nrows, zb, 0)
    nfull, rem = _RPS // nrows, _RPS % nrows
    for t in range(nfull):
        pltpu.sync_copy(zbuf, accum.at[pl.ds(s * _RPS + t * nrows, nrows)])
    if rem:
        pltpu.sync_copy(zbuf.at[pl.ds(0, rem)],
                        accum.at[pl.ds(s * _RPS + nfull * nrows, rem)])


@functools.partial(
    pl.kernel,
    out_type=jax.ShapeDtypeStruct((_NC, _N, _HID), _f32),
    mesh=_MESH,
    compiler_params=_SC_PARAMS,
    scratch_types=[
        pltpu.VMEM((_CH1, _HID), _f32),
        pltpu.VMEM((_CH1, _HID), _f32),
        pltpu.VMEM((_CH1,), jnp.int32),
        pltpu.VMEM((_CH1,), jnp.int32),
        pltpu.VMEM_SHARED((_N, _HID), _f32),
        pltpu.SemaphoreType.DMA,
        pltpu.SemaphoreType.DMA,
    ],
)
def _sc_segsum(e_hbm, idx_hbm, out_hbm, rows0, rows1, idx0, idx1,
               accum, insem0, insem1):
    c = lax.axis_index("c")
    s = lax.axis_index("s")
    _zero_accum(rows0, _CH1, accum, s)
    plsc.subcore_barrier()

    rows = (rows0, rows1)
    idxs = (idx0, idx1)
    insems = (insem0, insem1)
    base = (c * _NS + s) * _CPW1

    def in_copies(k, b):
        return (
            pltpu.make_async_copy(
                e_hbm.at[1, pl.ds(k * _CH1, _CH1), :], rows[b], insems[b]),
            pltpu.make_async_copy(idx_hbm.at[k], idxs[b], insems[b]),
        )

    for cp in in_copies(base, 0):
        cp.start()

    def step(q, b):
        k = base + q
        for cp in in_copies(k, b):
            cp.wait()

        @pl.when(q + 1 < _CPW1)
        def _():
            for cp in in_copies(k + 1, 1 - b):
                cp.start()

        pltpu.sync_copy(rows[b], accum.at[idxs[b]], add=True)

    def outer(t, _):
        step(t * 2, 0)
        step(t * 2 + 1, 1)
        return 0

    lax.fori_loop(0, _CPW1 // 2, outer, 0)

    plsc.subcore_barrier()
    sl = pl.ds(s * _RPS, _RPS)
    pltpu.sync_copy(accum.at[sl], out_hbm.at[c, sl, :])


@functools.partial(
    pl.kernel,
    out_type=jax.ShapeDtypeStruct((_NC, _N, _HID), _f32),
    mesh=_MESH,
    compiler_params=_SC_PARAMS,
    scratch_types=[
        pltpu.VMEM((_CH2, _HID), _f32),
        pltpu.VMEM((_CH2, _HID), _f32),
        pltpu.VMEM((_CH2, _HID), _f32),
        pltpu.VMEM((_CH2,), jnp.int32),
        pltpu.VMEM((_CH2,), jnp.int32),
        pltpu.VMEM((_CH2,), jnp.int32),
        pltpu.VMEM((_CH2,), jnp.int32),
        pltpu.VMEM_SHARED((_N, _HID), _f32),
        pltpu.SemaphoreType.DMA,
        pltpu.SemaphoreType.DMA,
        pltpu.SemaphoreType.DMA,
    ],
)
def _sc_gather_mul_segsum(wh_hbm, jdx_hbm, idst_hbm, tab_hbm, out_hbm,
                          wrows0, wrows1, grows, jdx0, jdx1,
                          ddx0, ddx1, accum, insem0, insem1, gsem):
    c = lax.axis_index("c")
    s = lax.axis_index("s")
    _zero_accum(wrows0, _CH2, accum, s)
    plsc.subcore_barrier()

    wrows = (wrows0, wrows1)
    jdxs = (jdx0, jdx1)
    ddxs = (ddx0, ddx1)
    insems = (insem0, insem1)
    base = (c * _NS + s) * _CPW2

    def in_copies(k, b):
        return (
            pltpu.make_async_copy(
                wh_hbm.at[pl.ds(k * _CH2, _CH2), :], wrows[b], insems[b]),
            pltpu.make_async_copy(jdx_hbm.at[k], jdxs[b], insems[b]),
            pltpu.make_async_copy(idst_hbm.at[k], ddxs[b], insems[b]),
        )

    def mul(b):
        def body(r, _):
            for qq in range(_HID // _L):
                sl2 = pl.ds(qq * _L, _L)
                wrows[b][r, sl2] = wrows[b][r, sl2] * grows[r, sl2]
            return 0

        lax.fori_loop(0, _CH2, body, 0)

    def step(q, b):
        k = base + q
        for cp in in_copies(k, b):
            cp.wait()

        @pl.when(q + 1 < _CPW2)
        def _():
            for cp in in_copies(k + 1, 1 - b):
                cp.start()

        pltpu.async_copy(tab_hbm.at[jdxs[b]], grows, gsem).wait()
        mul(b)
        pltpu.sync_copy(wrows[b], accum.at[ddxs[b]], add=True)

    for cp in in_copies(base, 0):
        cp.start()

    def outer(t, _):
        step(t * 2, 0)
        step(t * 2 + 1, 1)
        return 0

    lax.fori_loop(0, _CPW2 // 2, outer, 0)

    plsc.subcore_barrier()
    sl = pl.ds(s * _RPS, _RPS)
    pltpu.sync_copy(accum.at[sl], out_hbm.at[c, sl, :])


def _silu(x):
    return x * (1.0 / (1.0 + jnp.exp(-x)))


def _edge_mlp_body(x_ref, w0_ref, b0_ref, w1_ref, b1_ref, o_ref):
    h = jnp.dot(x_ref[...], w0_ref[...], preferred_element_type=_f32) + b0_ref[...]
    h = _silu(h)
    o_ref[...] = jnp.dot(h, w1_ref[...], preferred_element_type=_f32) + b1_ref[...]


def _edge_mlp(fea, w0, b0, w1, b1):
    blk = 2000
    grid = (_E // blk,)
    return pl.pallas_call(
        _edge_mlp_body,
        grid=grid,
        in_specs=[
            pl.BlockSpec((blk, 16), lambda g: (g, 0)),
            pl.BlockSpec((16, 128), lambda g: (0, 0)),
            pl.BlockSpec((128,), lambda g: (0,)),
            pl.BlockSpec((128, 128), lambda g: (0, 0)),
            pl.BlockSpec((128,), lambda g: (0,)),
        ],
        out_specs=pl.BlockSpec((blk, 128), lambda g: (g, 0)),
        out_shape=jax.ShapeDtypeStruct((_E, 128), _f32),
    )(fea, w0, b0, w1, b1)


def _vh_body(p_ref, w_ref, v_ref, vh_ref):
    v = p_ref[0] + p_ref[1]
    v_ref[...] = v
    vh_ref[...] = jnp.dot(v, w_ref[...], preferred_element_type=_f32)


def _v_vhull(parts, w):
    blk = 2000
    grid = (_N // blk,)
    return pl.pallas_call(
        _vh_body,
        grid=grid,
        in_specs=[
            pl.BlockSpec((_NC, blk, 128), lambda g: (0, g, 0)),
            pl.BlockSpec((128, 128), lambda g: (0, 0)),
        ],
        out_specs=[
            pl.BlockSpec((blk, 128), lambda g: (g, 0)),
            pl.BlockSpec((blk, 128), lambda g: (g, 0)),
        ],
        out_shape=[
            jax.ShapeDtypeStruct((_N, 128), _f32),
            jax.ShapeDtypeStruct((_N, 128), _f32),
        ],
    )(parts, w)


def _tail_body(v_ref, p_ref, w1h, b1h, w2h, b2h, wcv, wco, bc, wu, bu,
               wl0, bl0, wl1, bl1, wo, o_ref):
    oh = p_ref[0] + p_ref[1]
    oh = _silu(jnp.dot(oh, w1h[...], preferred_element_type=_f32) + b1h[...])
    oh = jnp.dot(oh, w2h[...], preferred_element_type=_f32) + b2h[...]
    v = v_ref[...]
    t = _silu(jnp.dot(v, wcv[...], preferred_element_type=_f32)
              + jnp.dot(oh, wco[...], preferred_element_type=_f32) + bc[...])
    u = jnp.dot(t, wu[...], preferred_element_type=_f32) + bu[...]
    u = _silu(jnp.dot(u, wl0[...], preferred_element_type=_f32) + bl0[...])
    u = _silu(jnp.dot(u, wl1[...], preferred_element_type=_f32) + bl1[...])
    o_ref[...] = jnp.dot(u, wo[...], preferred_element_type=_f32)


def _tail(v, parts, W_lin1h, b_lin1h, W_lin2h, b_lin2h, W_cat, b_cat,
          W_up, b_up, W_l0, b_l0, W_l1, b_l1, W_out):
    blk = 2000
    grid = (_N // blk,)
    wcv = W_cat[:128]
    wco = W_cat[128:]
    full = lambda *shape: pl.BlockSpec(shape, lambda g: (0,) * len(shape))
    return pl.pallas_call(
        _tail_body,
        grid=grid,
        in_specs=[
            pl.BlockSpec((blk, 128), lambda g: (g, 0)),
            pl.BlockSpec((_NC, blk, 128), lambda g: (0, g, 0)),
            full(128, 128), full(128,), full(128, 256), full(256,),
            full(128, 128), full(256, 128), full(128,),
            full(128, 256), full(256,),
            full(256, 256), full(256,), full(256, 256), full(256,),
            full(256, 128),
        ],
        out_specs=pl.BlockSpec((blk, 128), lambda g: (g, 0)),
        out_shape=jax.ShapeDtypeStruct((_N, 128), _f32),
    )(v, parts, W_lin1h, b_lin1h, W_lin2h, b_lin2h, wcv, wco, b_cat,
      W_up, b_up, W_l0, b_l0, W_l1, b_l1, W_out)


def kernel(e, i, fea_hull, edge_index_hull, W_lin_hull, W_mlp0, b_mlp0,
           W_mlp1, b_mlp1, W_lin1h, b_lin1h, W_lin2h, b_lin2h, W_cat, b_cat,
           W_up, b_up, W_l0, b_l0, W_l1, b_l1, W_out):
    i_r = i.astype(jnp.int32).reshape(_E // _CH1, _CH1)
    eih = edge_index_hull.astype(jnp.int32).reshape(2, _E // _CH2, _CH2)

    v_parts = _sc_segsum(e, i_r)
    W_h = _edge_mlp(fea_hull, W_mlp0, b_mlp0, W_mlp1, b_mlp1)
    v, v_hull = _v_vhull(v_parts, W_lin_hull)
    oh_parts = _sc_gather_mul_segsum(W_h, eih[0], eih[1], v_hull)
    return _tail(v, oh_parts, W_lin1h, b_lin1h, W_lin2h, b_lin2h,
                 W_cat, b_cat, W_up, b_up, W_l0, b_l0, W_l1, b_l1, W_out)


# speedup vs baseline: 1.4511x; 1.0673x over previous
import functools

import jax
import jax.numpy as jnp
from jax import lax
from jax.experimental import pallas as pl
from jax.experimental.pallas import tpu as pltpu
from jax.experimental.pallas import tpu_sc as plsc

_N = 10000
_E = 320000
_HID = 128
_NC, _NS, _L = 2, 16, 16
_NW = _NC * _NS
_CH1 = 100
_CPW1 = _E // _CH1 // _NW
_CH2 = 100
_CPW2 = _E // _CH2 // _NW
_RPS = _N // _NS

_MESH = plsc.VectorSubcoreMesh(
    core_axis_name="c", subcore_axis_name="s", num_cores=_NC, num_subcores=_NS)
_SC_PARAMS = pltpu.CompilerParams(use_tc_tiling_on_sc=False)

_f32 = jnp.float32


def _zero_accum(zbuf, nrows, accum, s):
    zeros = jnp.zeros((_L,), _f32)

    def zb(r, _):
        for q in range(_HID // _L):
            zbuf[r, pl.ds(q * _L, _L)] = zeros
        return 0

    lax.fori_loop(0, nrows, zb, 0)
    nfull, rem = _RPS // nrows, _RPS % nrows
    for t in range(nfull):
        pltpu.sync_copy(zbuf, accum.at[pl.ds(s * _RPS + t * nrows, nrows)])
    if rem:
        pltpu.sync_copy(zbuf.at[pl.ds(0, rem)],
                        accum.at[pl.ds(s * _RPS + nfull * nrows, rem)])


@functools.partial(
    pl.kernel,
    out_type=jax.ShapeDtypeStruct((_NC, _N, _HID), _f32),
    mesh=_MESH,
    compiler_params=_SC_PARAMS,
    scratch_types=[
        pltpu.VMEM((_CH1, _HID), _f32),
        pltpu.VMEM((_CH1, _HID), _f32),
        pltpu.VMEM((_CH1,), jnp.int32),
        pltpu.VMEM((_CH1,), jnp.int32),
        pltpu.VMEM_SHARED((_N, _HID), _f32),
        pltpu.SemaphoreType.DMA,
        pltpu.SemaphoreType.DMA,
    ],
)
def _sc_segsum(e_hbm, idx_hbm, out_hbm, rows0, rows1, idx0, idx1,
               accum, insem0, insem1):
    c = lax.axis_index("c")
    s = lax.axis_index("s")
    _zero_accum(rows0, _CH1, accum, s)
    plsc.subcore_barrier()

    rows = (rows0, rows1)
    idxs = (idx0, idx1)
    insems = (insem0, insem1)
    base = (c * _NS + s) * _CPW1

    def in_copies(k, b):
        return (
            pltpu.make_async_copy(
                e_hbm.at[1, pl.ds(k * _CH1, _CH1), :], rows[b], insems[b]),
            pltpu.make_async_copy(idx_hbm.at[k], idxs[b], insems[b]),
        )

    for cp in in_copies(base, 0):
        cp.start()

    def step(q, b):
        k = base + q
        for cp in in_copies(k, b):
            cp.wait()

        @pl.when(q + 1 < _CPW1)
        def _():
            for cp in in_copies(k + 1, 1 - b):
                cp.start()

        pltpu.sync_copy(rows[b], accum.at[idxs[b]], add=True)

    def outer(t, _):
        step(t * 2, 0)
        step(t * 2 + 1, 1)
        return 0

    lax.fori_loop(0, _CPW1 // 2, outer, 0)

    plsc.subcore_barrier()
    sl = pl.ds(s * _RPS, _RPS)
    pltpu.sync_copy(accum.at[sl], out_hbm.at[c, sl, :])


@functools.partial(
    pl.kernel,
    out_type=jax.ShapeDtypeStruct((_NC, _N, _HID), _f32),
    mesh=_MESH,
    compiler_params=_SC_PARAMS,
    scratch_types=[
        pltpu.VMEM((_CH2, _HID), _f32),
        pltpu.VMEM((_CH2, _HID), _f32),
        pltpu.VMEM((_CH2, _HID), _f32),
        pltpu.VMEM((_CH2,), jnp.int32),
        pltpu.VMEM((_CH2,), jnp.int32),
        pltpu.VMEM((_CH2,), jnp.int32),
        pltpu.VMEM((_CH2,), jnp.int32),
        pltpu.VMEM_SHARED((_N, _HID), _f32),
        pltpu.SemaphoreType.DMA,
        pltpu.SemaphoreType.DMA,
        pltpu.SemaphoreType.DMA,
        pltpu.SemaphoreType.DMA,
        pltpu.SemaphoreType.DMA,
    ],
)
def _sc_gather_mul_segsum(wh_hbm, jdx_hbm, idst_hbm, tab_hbm, out_hbm,
                          wrows, grows0, grows1, jdx0, jdx1,
                          ddx0, ddx1, accum, insemw, insemi0, insemi1,
                          gsem0, gsem1):
    c = lax.axis_index("c")
    s = lax.axis_index("s")
    _zero_accum(wrows, _CH2, accum, s)
    plsc.subcore_barrier()

    grows = (grows0, grows1)
    jdxs = (jdx0, jdx1)
    ddxs = (ddx0, ddx1)
    insemis = (insemi0, insemi1)
    gsems = (gsem0, gsem1)
    base = (c * _NS + s) * _CPW2

    def idx_copies(k, b):
        return (
            pltpu.make_async_copy(jdx_hbm.at[k], jdxs[b], insemis[b]),
            pltpu.make_async_copy(idst_hbm.at[k], ddxs[b], insemis[b]),
        )

    def wh_copy(k):
        return pltpu.make_async_copy(
            wh_hbm.at[pl.ds(k * _CH2, _CH2), :], wrows, insemw)

    def gath(b):
        return pltpu.make_async_copy(tab_hbm.at[jdxs[b]], grows[b], gsems[b])

    def mul(b):
        def body(r, _):
            for qq in range(_HID // _L):
                sl2 = pl.ds(qq * _L, _L)
                wrows[r, sl2] = wrows[r, sl2] * grows[b][r, sl2]
            return 0

        lax.fori_loop(0, _CH2, body, 0)

    def step(q, b):
        k = base + q

        @pl.when(q + 1 < _CPW2)
        def _():
            for cp in idx_copies(k + 1, 1 - b):
                cp.wait()

        gath(b).wait()
        wh_copy(k).wait()

        @pl.when(q + 1 < _CPW2)
        def _():
            pltpu.async_copy(tab_hbm.at[jdxs[1 - b]], grows[1 - b],
                             gsems[1 - b])

        mul(b)
        pltpu.sync_copy(wrows, accum.at[ddxs[b]], add=True)

        @pl.when(q + 2 < _CPW2)
        def _():
            for cp in idx_copies(k + 2, b):
                cp.start()

        @pl.when(q + 1 < _CPW2)
        def _():
            wh_copy(k + 1).start()

    for cp in idx_copies(base, 0):
        cp.start()
    for cp in idx_copies(base, 0):
        cp.wait()
    pltpu.async_copy(tab_hbm.at[jdxs[0]], grows[0], gsems[0])
    for cp in idx_copies(base + 1, 1):
        cp.start()
    wh_copy(base).start()

    def outer(t, _):
        step(t * 2, 0)
        step(t * 2 + 1, 1)
        return 0

    lax.fori_loop(0, _CPW2 // 2, outer, 0)

    plsc.subcore_barrier()
    sl = pl.ds(s * _RPS, _RPS)
    pltpu.sync_copy(accum.at[sl], out_hbm.at[c, sl, :])


def _silu(x):
    return x * (1.0 / (1.0 + jnp.exp(-x)))


def _edge_mlp_body(x_ref, w0_ref, b0_ref, w1_ref, b1_ref, o_ref):
    h = jnp.dot(x_ref[...], w0_ref[...], preferred_element_type=_f32) + b0_ref[...]
    h = _silu(h)
    o_ref[...] = jnp.dot(h, w1_ref[...], preferred_element_type=_f32) + b1_ref[...]


def _edge_mlp(fea, w0, b0, w1, b1):
    blk = 2000
    grid = (_E // blk,)
    return pl.pallas_call(
        _edge_mlp_body,
        grid=grid,
        in_specs=[
            pl.BlockSpec((blk, 16), lambda g: (g, 0)),
            pl.BlockSpec((16, 128), lambda g: (0, 0)),
            pl.BlockSpec((128,), lambda g: (0,)),
            pl.BlockSpec((128, 128), lambda g: (0, 0)),
            pl.BlockSpec((128,), lambda g: (0,)),
        ],
        out_specs=pl.BlockSpec((blk, 128), lambda g: (g, 0)),
        out_shape=jax.ShapeDtypeStruct((_E, 128), _f32),
    )(fea, w0, b0, w1, b1)


def _vh_body(p_ref, w_ref, v_ref, vh_ref):
    v = p_ref[0] + p_ref[1]
    v_ref[...] = v
    vh_ref[...] = jnp.dot(v, w_ref[...], preferred_element_type=_f32)


def _v_vhull(parts, w):
    blk = 2000
    grid = (_N // blk,)
    return pl.pallas_call(
        _vh_body,
        grid=grid,
        in_specs=[
            pl.BlockSpec((_NC, blk, 128), lambda g: (0, g, 0)),
            pl.BlockSpec((128, 128), lambda g: (0, 0)),
        ],
        out_specs=[
            pl.BlockSpec((blk, 128), lambda g: (g, 0)),
            pl.BlockSpec((blk, 128), lambda g: (g, 0)),
        ],
        out_shape=[
            jax.ShapeDtypeStruct((_N, 128), _f32),
            jax.ShapeDtypeStruct((_N, 128), _f32),
        ],
    )(parts, w)


def _tail_body(v_ref, p_ref, w1h, b1h, w2h, b2h, wcv, wco, bc, wu, bu,
               wl0, bl0, wl1, bl1, wo, o_ref):
    oh = p_ref[0] + p_ref[1]
    oh = _silu(jnp.dot(oh, w1h[...], preferred_element_type=_f32) + b1h[...])
    oh = jnp.dot(oh, w2h[...], preferred_element_type=_f32) + b2h[...]
    v = v_ref[...]
    t = _silu(jnp.dot(v, wcv[...], preferred_element_type=_f32)
              + jnp.dot(oh, wco[...], preferred_element_type=_f32) + bc[...])
    u = jnp.dot(t, wu[...], preferred_element_type=_f32) + bu[...]
    u = _silu(jnp.dot(u, wl0[...], preferred_element_type=_f32) + bl0[...])
    u = _silu(jnp.dot(u, wl1[...], preferred_element_type=_f32) + bl1[...])
    o_ref[...] = jnp.dot(u, wo[...], preferred_element_type=_f32)


def _tail(v, parts, W_lin1h, b_lin1h, W_lin2h, b_lin2h, W_cat, b_cat,
          W_up, b_up, W_l0, b_l0, W_l1, b_l1, W_out):
    blk = 2000
    grid = (_N // blk,)
    wcv = W_cat[:128]
    wco = W_cat[128:]
    full = lambda *shape: pl.BlockSpec(shape, lambda g: (0,) * len(shape))
    return pl.pallas_call(
        _tail_body,
        grid=grid,
        in_specs=[
            pl.BlockSpec((blk, 128), lambda g: (g, 0)),
            pl.BlockSpec((_NC, blk, 128), lambda g: (0, g, 0)),
            full(128, 128), full(128,), full(128, 256), full(256,),
            full(128, 128), full(256, 128), full(128,),
            full(128, 256), full(256,),
            full(256, 256), full(256,), full(256, 256), full(256,),
            full(256, 128),
        ],
        out_specs=pl.BlockSpec((blk, 128), lambda g: (g, 0)),
        out_shape=jax.ShapeDtypeStruct((_N, 128), _f32),
    )(v, parts, W_lin1h, b_lin1h, W_lin2h, b_lin2h, wcv, wco, b_cat,
      W_up, b_up, W_l0, b_l0, W_l1, b_l1, W_out)


def kernel(e, i, fea_hull, edge_index_hull, W_lin_hull, W_mlp0, b_mlp0,
           W_mlp1, b_mlp1, W_lin1h, b_lin1h, W_lin2h, b_lin2h, W_cat, b_cat,
           W_up, b_up, W_l0, b_l0, W_l1, b_l1, W_out):
    i_r = i.astype(jnp.int32).reshape(_E // _CH1, _CH1)
    eih = edge_index_hull.astype(jnp.int32).reshape(2, _E // _CH2, _CH2)

    v_parts = _sc_segsum(e, i_r)
    W_h = _edge_mlp(fea_hull, W_mlp0, b_mlp0, W_mlp1, b_mlp1)
    v, v_hull = _v_vhull(v_parts, W_lin_hull)
    oh_parts = _sc_gather_mul_segsum(W_h, eih[0], eih[1], v_hull)
    return _tail(v, oh_parts, W_lin1h, b_lin1h, W_lin2h, b_lin2h,
                 W_cat, b_cat, W_up, b_up, W_l0, b_l0, W_l1, b_l1, W_out)


# speedup vs baseline: 1.4525x; 1.0010x over previous
import functools

import jax
import jax.numpy as jnp
from jax import lax
from jax.experimental import pallas as pl
from jax.experimental.pallas import tpu as pltpu
from jax.experimental.pallas import tpu_sc as plsc

_N = 10000
_E = 320000
_HID = 128
_NC, _NS, _L = 2, 16, 16
_NW = _NC * _NS
_CH1 = 100
_CPW1 = _E // _CH1 // _NW
_CH2 = 100
_CPW2 = _E // _CH2 // _NW
_RPS = _N // _NS

_MESH = plsc.VectorSubcoreMesh(
    core_axis_name="c", subcore_axis_name="s", num_cores=_NC, num_subcores=_NS)
_SC_PARAMS = pltpu.CompilerParams(use_tc_tiling_on_sc=False)

_f32 = jnp.float32


def _zero_accum(zbuf, nrows, accum, s):
    zeros = jnp.zeros((_L,), _f32)

    def zb(r, _):
        for q in range(_HID // _L):
            zbuf[r, pl.ds(q * _L, _L)] = zeros
        return 0

    lax.fori_loop(0, nrows, zb, 0)
    nfull, rem = _RPS // nrows, _RPS % nrows
    for t in range(nfull):
        pltpu.sync_copy(zbuf, accum.at[pl.ds(s * _RPS + t * nrows, nrows)])
    if rem:
        pltpu.sync_copy(zbuf.at[pl.ds(0, rem)],
                        accum.at[pl.ds(s * _RPS + nfull * nrows, rem)])


@functools.partial(
    pl.kernel,
    out_type=jax.ShapeDtypeStruct((_NC, _N, _HID), _f32),
    mesh=_MESH,
    compiler_params=_SC_PARAMS,
    scratch_types=[
        pltpu.VMEM((_CH1, _HID), _f32),
        pltpu.VMEM((_CH1, _HID), _f32),
        pltpu.VMEM((_CH1,), jnp.int32),
        pltpu.VMEM((_CH1,), jnp.int32),
        pltpu.VMEM_SHARED((_N, _HID), _f32),
        pltpu.SemaphoreType.DMA,
        pltpu.SemaphoreType.DMA,
    ],
)
def _sc_segsum(e_hbm, idx_hbm, out_hbm, rows0, rows1, idx0, idx1,
               accum, insem0, insem1):
    c = lax.axis_index("c")
    s = lax.axis_index("s")
    _zero_accum(rows0, _CH1, accum, s)
    plsc.subcore_barrier()

    rows = (rows0, rows1)
    idxs = (idx0, idx1)
    insems = (insem0, insem1)
    base = (c * _NS + s) * _CPW1

    def in_copies(k, b):
        return (
            pltpu.make_async_copy(
                e_hbm.at[1, pl.ds(k * _CH1, _CH1), :], rows[b], insems[b]),
            pltpu.make_async_copy(idx_hbm.at[k], idxs[b], insems[b]),
        )

    for cp in in_copies(base, 0):
        cp.start()

    def step(q, b):
        k = base + q
        for cp in in_copies(k, b):
            cp.wait()

        @pl.when(q + 1 < _CPW1)
        def _():
            for cp in in_copies(k + 1, 1 - b):
                cp.start()

        pltpu.sync_copy(rows[b], accum.at[idxs[b]], add=True)

    def outer(t, _):
        step(t * 2, 0)
        step(t * 2 + 1, 1)
        return 0

    lax.fori_loop(0, _CPW1 // 2, outer, 0)

    plsc.subcore_barrier()
    sl = pl.ds(s * _RPS, _RPS)
    pltpu.sync_copy(accum.at[sl], out_hbm.at[c, sl, :])


@functools.partial(
    pl.kernel,
    out_type=jax.ShapeDtypeStruct((_NC, _N, _HID), _f32),
    mesh=_MESH,
    compiler_params=_SC_PARAMS,
    scratch_types=[
        pltpu.VMEM((_CH2, _HID), _f32),
        pltpu.VMEM((_CH2, _HID), _f32),
        pltpu.VMEM((_CH2, _HID), _f32),
        pltpu.VMEM((_CH2,), jnp.int32),
        pltpu.VMEM((_CH2,), jnp.int32),
        pltpu.VMEM((_CH2,), jnp.int32),
        pltpu.VMEM((_CH2,), jnp.int32),
        pltpu.VMEM_SHARED((_N, _HID), _f32),
        pltpu.SemaphoreType.DMA,
        pltpu.SemaphoreType.DMA,
        pltpu.SemaphoreType.DMA,
        pltpu.SemaphoreType.DMA,
        pltpu.SemaphoreType.DMA,
    ],
)
def _sc_gather_mul_segsum(wh_hbm, jdx_hbm, idst_hbm, tab_hbm, out_hbm,
                          wrows, grows0, grows1, jdx0, jdx1,
                          ddx0, ddx1, accum, insemw, insemi0, insemi1,
                          gsem0, gsem1):
    c = lax.axis_index("c")
    s = lax.axis_index("s")
    _zero_accum(wrows, _CH2, accum, s)
    plsc.subcore_barrier()

    grows = (grows0, grows1)
    jdxs = (jdx0, jdx1)
    ddxs = (ddx0, ddx1)
    insemis = (insemi0, insemi1)
    gsems = (gsem0, gsem1)
    base = (c * _NS + s) * _CPW2

    def idx_copies(k, b):
        return (
            pltpu.make_async_copy(jdx_hbm.at[k], jdxs[b], insemis[b]),
            pltpu.make_async_copy(idst_hbm.at[k], ddxs[b], insemis[b]),
        )

    def wh_copy(k):
        return pltpu.make_async_copy(
            wh_hbm.at[pl.ds(k * _CH2, _CH2), :], wrows, insemw)

    def gath(b):
        return pltpu.make_async_copy(tab_hbm.at[jdxs[b]], grows[b], gsems[b])

    def mul(b):
        def body(r, _):
            for qq in range(_HID // _L):
                sl2 = pl.ds(qq * _L, _L)
                wrows[r, sl2] = wrows[r, sl2] * grows[b][r, sl2]
            return 0

        lax.fori_loop(0, _CH2, body, 0)

    def step(q, b):
        k = base + q

        @pl.when(q + 1 < _CPW2)
        def _():
            for cp in idx_copies(k + 1, 1 - b):
                cp.wait()

        gath(b).wait()
        wh_copy(k).wait()

        @pl.when(q + 1 < _CPW2)
        def _():
            pltpu.async_copy(tab_hbm.at[jdxs[1 - b]], grows[1 - b],
                             gsems[1 - b])

        mul(b)
        pltpu.sync_copy(wrows, accum.at[ddxs[b]], add=True)

        @pl.when(q + 2 < _CPW2)
        def _():
            for cp in idx_copies(k + 2, b):
                cp.start()

        @pl.when(q + 1 < _CPW2)
        def _():
            wh_copy(k + 1).start()

    for cp in idx_copies(base, 0):
        cp.start()
    for cp in idx_copies(base, 0):
        cp.wait()
    pltpu.async_copy(tab_hbm.at[jdxs[0]], grows[0], gsems[0])
    for cp in idx_copies(base + 1, 1):
        cp.start()
    wh_copy(base).start()

    def outer(t, _):
        step(t * 2, 0)
        step(t * 2 + 1, 1)
        return 0

    lax.fori_loop(0, _CPW2 // 2, outer, 0)

    plsc.subcore_barrier()
    sl = pl.ds(s * _RPS, _RPS)
    pltpu.sync_copy(accum.at[sl], out_hbm.at[c, sl, :])


def _silu(x):
    return x * (1.0 / (1.0 + jnp.exp(-x)))


def _edge_mlp_body(x_ref, w0_ref, b0_ref, w1_ref, b1_ref, o_ref):
    h = jnp.dot(x_ref[...], w0_ref[...], preferred_element_type=_f32) + b0_ref[...]
    h = _silu(h)
    o_ref[...] = jnp.dot(h, w1_ref[...], preferred_element_type=_f32) + b1_ref[...]


def _edge_mlp(fea, w0, b0, w1, b1):
    blk = 2000
    grid = (_E // blk,)
    return pl.pallas_call(
        _edge_mlp_body,
        grid=grid,
        in_specs=[
            pl.BlockSpec((blk, 16), lambda g: (g, 0)),
            pl.BlockSpec((16, 128), lambda g: (0, 0)),
            pl.BlockSpec((128,), lambda g: (0,)),
            pl.BlockSpec((128, 128), lambda g: (0, 0)),
            pl.BlockSpec((128,), lambda g: (0,)),
        ],
        out_specs=pl.BlockSpec((blk, 128), lambda g: (g, 0)),
        out_shape=jax.ShapeDtypeStruct((_E, 128), _f32),
    )(fea, w0, b0, w1, b1)


def _vh_body(p_ref, w_ref, v_ref, vh_ref):
    v = p_ref[0] + p_ref[1]
    v_ref[...] = v
    vh_ref[...] = jnp.dot(v, w_ref[...], preferred_element_type=_f32)


def _v_vhull(parts, w):
    blk = 2000
    grid = (_N // blk,)
    return pl.pallas_call(
        _vh_body,
        grid=grid,
        in_specs=[
            pl.BlockSpec((_NC, blk, 128), lambda g: (0, g, 0)),
            pl.BlockSpec((128, 128), lambda g: (0, 0)),
        ],
        out_specs=[
            pl.BlockSpec((blk, 128), lambda g: (g, 0)),
            pl.BlockSpec((blk, 128), lambda g: (g, 0)),
        ],
        out_shape=[
            jax.ShapeDtypeStruct((_N, 128), _f32),
            jax.ShapeDtypeStruct((_N, 128), _f32),
        ],
    )(parts, w)


def _tail_body(v_ref, p_ref, w1h, b1h, w2h, b2h, wcv, wco, bc, wu, bu,
               wl0, bl0, wl1, bl1, wo, o_ref):
    oh = p_ref[0] + p_ref[1]
    oh = _silu(jnp.dot(oh, w1h[...], preferred_element_type=_f32) + b1h[...])
    oh = jnp.dot(oh, w2h[...], preferred_element_type=_f32) + b2h[...]
    v = v_ref[...]
    t = _silu(jnp.dot(v, wcv[...], preferred_element_type=_f32)
              + jnp.dot(oh, wco[...], preferred_element_type=_f32) + bc[...])
    u = jnp.dot(t, wu[...], preferred_element_type=_f32) + bu[...]
    u = _silu(jnp.dot(u, wl0[...], preferred_element_type=_f32) + bl0[...])
    u = _silu(jnp.dot(u, wl1[...], preferred_element_type=_f32) + bl1[...])
    o_ref[...] = jnp.dot(u, wo[...], preferred_element_type=_f32)


def _tail(v, parts, W_lin1h, b_lin1h, W_lin2h, b_lin2h, W_cat, b_cat,
          W_up, b_up, W_l0, b_l0, W_l1, b_l1, W_out):
    blk = 2000
    grid = (_N // blk,)
    wcv = W_cat[:128]
    wco = W_cat[128:]
    full = lambda *shape: pl.BlockSpec(shape, lambda g: (0,) * len(shape))
    return pl.pallas_call(
        _tail_body,
        grid=grid,
        in_specs=[
            pl.BlockSpec((blk, 128), lambda g: (g, 0)),
            pl.BlockSpec((_NC, blk, 128), lambda g: (0, g, 0)),
            full(128, 128), full(128,), full(128, 256), full(256,),
            full(128, 128), full(256, 128), full(128,),
            full(128, 256), full(256,),
            full(256, 256), full(256,), full(256, 256), full(256,),
            full(256, 128),
        ],
        out_specs=pl.BlockSpec((blk, 128), lambda g: (g, 0)),
        out_shape=jax.ShapeDtypeStruct((_N, 128), _f32),
    )(v, parts, W_lin1h, b_lin1h, W_lin2h, b_lin2h, wcv, wco, b_cat,
      W_up, b_up, W_l0, b_l0, W_l1, b_l1, W_out)


def kernel(e, i, fea_hull, edge_index_hull, W_lin_hull, W_mlp0, b_mlp0,
           W_mlp1, b_mlp1, W_lin1h, b_lin1h, W_lin2h, b_lin2h, W_cat, b_cat,
           W_up, b_up, W_l0, b_l0, W_l1, b_l1, W_out):
    i_r = i.astype(jnp.int32).reshape(_E // _CH1, _CH1)
    eih = edge_index_hull.astype(jnp.int32).reshape(2, _E // _CH2, _CH2)

    W_h = _edge_mlp(fea_hull, W_mlp0, b_mlp0, W_mlp1, b_mlp1)
    v_parts = _sc_segsum(e, i_r)
    v, v_hull = _v_vhull(v_parts, W_lin_hull)
    oh_parts = _sc_gather_mul_segsum(W_h, eih[0], eih[1], v_hull)
    return _tail(v, oh_parts, W_lin1h, b_lin1h, W_lin2h, b_lin2h,
                 W_cat, b_cat, W_up, b_up, W_l0, b_l0, W_l1, b_l1, W_out)
